# baseline (speedup 1.0000x reference)
import functools

import jax
import jax.numpy as jnp
from jax import lax
from jax.experimental import pallas as pl
from jax.experimental.pallas import tpu as pltpu
from jax.experimental.pallas import tpu_sc as plsc

N = 10000
E = 160000
D = 128
NB = 32

NCORE = 2
NSUB = 16
EPT = E // NSUB
CH = 80
NCH = EPT // CH
RPT = 624
LAST = N - (NSUB - 1) * RPT


def _mlp_body(x_ref, ws1, bs1, ws2, bs2, wv1, bv1, wv2, bv2, os_ref, ov_ref):
    x = x_ref[...]
    hs = jax.nn.silu(jnp.dot(x, ws1[...], preferred_element_type=jnp.float32) + bs1[...])
    os_ref[...] = jnp.dot(hs, ws2[...], preferred_element_type=jnp.float32) + bs2[...]
    hv = jax.nn.silu(jnp.dot(x, wv1[...], preferred_element_type=jnp.float32) + bv1[...])
    ov_ref[...] = jnp.dot(hv, wv2[...], preferred_element_type=jnp.float32) + bv2[...]


def _vt_body(x_ref, o_ref):
    o_ref[...] = x_ref[0]


def _vec_transpose(outv):
    BN = 2000
    out = pl.pallas_call(
        _vt_body,
        grid=(N // BN, 3),
        in_specs=[pl.BlockSpec((1, BN, D), lambda i, j: (j, i, 0))],
        out_specs=pl.BlockSpec((BN, D), lambda i, j: (i, j)),
        out_shape=jax.ShapeDtypeStruct((N, 3 * D), jnp.float32),
    )(outv.reshape(3, N, D))
    return out.reshape(N, 3, D)


def _edge_mlps(el, Ws1, bs1, Ws2, bs2, Wv1, bv1, Wv2, bv2):
    BE = 16000
    full = lambda shp: pl.BlockSpec(shp, lambda i: (0,) * len(shp))
    return pl.pallas_call(
        _mlp_body,
        grid=(E // BE,),
        in_specs=[
            pl.BlockSpec((BE, NB), lambda i: (i, 0)),
            full((NB, D)), full((1, D)), full((D, D)), full((1, D)),
            full((NB, D)), full((1, D)), full((D, D)), full((1, D)),
        ],
        out_specs=[pl.BlockSpec((BE, D), lambda i: (i, 0)),
                   pl.BlockSpec((BE, D), lambda i: (i, 0))],
        out_shape=(jax.ShapeDtypeStruct((E, D), jnp.float32),
                   jax.ShapeDtypeStruct((E, D), jnp.float32)),
    )(el, Ws1, bs1.reshape(1, D), Ws2, bs2.reshape(1, D),
      Wv1, bv1.reshape(1, D), Wv2, bv2.reshape(1, D))


def _sc_body(tabs, tabv, ws, wv, eix, zrows, outs, outv,
             accum, icA, icB, irA, irB, gbA, gbB, rA, wA, rB, wB,
             sIA, sIB, sAg, sAw, sBg, sBw):
    c = lax.axis_index("c")
    s = lax.axis_index("s")

    def idx_issue(g, icbuf, irbuf, si):
        base = s * EPT + g * CH
        pltpu.async_copy(eix.at[pl.ds(E + base, CH)], icbuf, si)
        pltpu.async_copy(eix.at[pl.ds(base, CH)], irbuf, si)

    def idx_wait(g, icbuf, irbuf, si):
        base = s * EPT + g * CH
        pltpu.make_async_copy(eix.at[pl.ds(E + base, CH)], icbuf, si).wait()
        pltpu.make_async_copy(eix.at[pl.ds(base, CH)], irbuf, si).wait()

    def run_phase(tab, mult, chb, wref):
        wbase = s * EPT

        def bias(icbuf, gbuf):
            for k in range(CH // 16):
                gbuf[pl.ds(k * 16, 16)] = icbuf[pl.ds(k * 16, 16)] * mult + chb

        def data_issue(g, gbuf, rbuf, wbuf, sg, sw):
            pltpu.async_copy(tab.at[gbuf], rbuf, sg)
            pltpu.async_copy(wref.at[pl.ds(wbase + g * CH, CH)], wbuf, sw)

        def data_wait(g, gbuf, rbuf, wbuf, sg, sw):
            pltpu.make_async_copy(tab.at[gbuf], rbuf, sg).wait()
            pltpu.make_async_copy(wref.at[pl.ds(wbase + g * CH, CH)], wbuf, sw).wait()

        def mul_scatter(irbuf, rbuf, wbuf, pre_issue=None):
            def mrow(r, _):
                for j in range(D // 16):
                    sl = pl.ds(j * 16, 16)
                    rbuf[r, sl] = rbuf[r, sl] * wbuf[r, sl]
                return 0
            lax.fori_loop(0, CH, mrow, 0)
            rows16 = [irbuf[pl.ds(k * 16, 16)] for k in range(CH // 16)]
            if pre_issue is not None:
                pre_issue()
            for k in range(CH // 16):
                pltpu.sync_copy(rbuf.at[pl.ds(k * 16, 16)],
                                accum.at[rows16[k]], add=True)

        idx_issue(0, icA, irA, sIA)
        idx_wait(0, icA, irA, sIA)
        bias(icA, gbA)
        data_issue(0, gbA, rA, wA, sAg, sAw)
        idx_issue(1, icB, irB, sIB)

        def chunk_pair(i, _):
            g0 = 2 * i
            idx_wait(g0 + 1, icB, irB, sIB)
            bias(icB, gbB)
            data_issue(g0 + 1, gbB, rB, wB, sBg, sBw)

            data_wait(g0, gbA, rA, wA, sAg, sAw)
            mul_scatter(irA, rA, wA,
                        pre_issue=lambda: idx_issue(g0 + 2, icA, irA, sIA))

            idx_wait(g0 + 2, icA, irA, sIA)
            bias(icA, gbA)
            data_issue(g0 + 2, gbA, rA, wA, sAg, sAw)

            data_wait(g0 + 1, gbB, rB, wB, sBg, sBw)

            def refill_b():
                @pl.when(i < NCH // 2 - 1)
                def _():
                    idx_issue(g0 + 3, icB, irB, sIB)

            mul_scatter(irB, rB, wB, pre_issue=refill_b)
            return 0
        lax.fori_loop(0, NCH // 2, chunk_pair, 0)

        data_wait(NCH - 1, gbA, rA, wA, sAg, sAw)
        mul_scatter(irA, rA, wA)

    def writeout_s():
        @pl.when(s < NSUB - 1)
        def _():
            pltpu.sync_copy(accum.at[pl.ds(s * RPT, RPT)],
                            outs.at[pl.ds(s * RPT, RPT)])

        @pl.when(s == NSUB - 1)
        def _():
            pltpu.sync_copy(accum.at[pl.ds((NSUB - 1) * RPT, LAST)],
                            outs.at[pl.ds((NSUB - 1) * RPT, LAST)])

    def writeout_v(ch):
        dbase = ch * N

        @pl.when(s < NSUB - 1)
        def _():
            pltpu.sync_copy(accum.at[pl.ds(s * RPT, RPT)],
                            outv.at[pl.ds(dbase + s * RPT, RPT)])

        @pl.when(s == NSUB - 1)
        def _():
            pltpu.sync_copy(accum.at[pl.ds((NSUB - 1) * RPT, LAST)],
                            outv.at[pl.ds(dbase + (NSUB - 1) * RPT, LAST)])

    def zero_accum():
        @pl.when(s < NSUB - 1)
        def _():
            pltpu.sync_copy(zrows.at[pl.ds(0, RPT)], accum.at[pl.ds(s * RPT, RPT)])

        @pl.when(s == NSUB - 1)
        def _():
            pltpu.sync_copy(zrows, accum.at[pl.ds((NSUB - 1) * RPT, LAST)])

    zero_accum()
    plsc.subcore_barrier()

    @pl.when(c == 0)
    def _():
        run_phase(tabs, 1, 0 * c, ws)

    @pl.when(c == 1)
    def _():
        run_phase(tabv, 3, 0 * c + 1, wv)

    plsc.subcore_barrier()

    @pl.when(c == 0)
    def _():
        writeout_s()

    @pl.when(c == 1)
    def _():
        writeout_v(0 * c + 1)

    plsc.subcore_barrier()

    zero_accum()
    plsc.subcore_barrier()

    run_phase(tabv, 3, 2 * c, wv)

    plsc.subcore_barrier()
    writeout_v(2 * c)
    plsc.subcore_barrier()


_sc_gather_scatter = functools.partial(
    pl.kernel,
    out_type=(jax.ShapeDtypeStruct((N, D), jnp.float32),
              jax.ShapeDtypeStruct((3 * N, D), jnp.float32)),
    mesh=plsc.VectorSubcoreMesh(core_axis_name="c", subcore_axis_name="s"),
    scratch_types=[
        pltpu.VMEM_SHARED((N, D), jnp.float32),
        pltpu.VMEM((CH,), jnp.int32),
        pltpu.VMEM((CH,), jnp.int32),
        pltpu.VMEM((CH,), jnp.int32),
        pltpu.VMEM((CH,), jnp.int32),
        pltpu.VMEM((CH,), jnp.int32),
        pltpu.VMEM((CH,), jnp.int32),
        pltpu.VMEM((CH, D), jnp.float32),
        pltpu.VMEM((CH, D), jnp.float32),
        pltpu.VMEM((CH, D), jnp.float32),
        pltpu.VMEM((CH, D), jnp.float32),
        pltpu.SemaphoreType.DMA,
        pltpu.SemaphoreType.DMA,
        pltpu.SemaphoreType.DMA,
        pltpu.SemaphoreType.DMA,
        pltpu.SemaphoreType.DMA,
        pltpu.SemaphoreType.DMA,
    ],
)(_sc_body)


def kernel(scalar, vector, edge_sh, edge_length, edge_index, Ws1, bs1, Ws2, bs2, Wv1, bv1, Wv2, bv2):
    del edge_sh
    ws, wv = _edge_mlps(edge_length, Ws1, bs1, Ws2, bs2, Wv1, bv1, Wv2, bv2)
    eix = edge_index.astype(jnp.int32).reshape(2 * E)
    tabv = vector.reshape(N * 3, D)
    zrows = jnp.zeros((LAST, D), jnp.float32)
    out_scalar, outv = _sc_gather_scatter(scalar, tabv, ws, wv, eix, zrows)
    return out_scalar, _vec_transpose(outv)


# speedup vs baseline: 1.0639x; 1.0639x over previous
import functools

import jax
import jax.numpy as jnp
from jax import lax
from jax.experimental import pallas as pl
from jax.experimental.pallas import tpu as pltpu
from jax.experimental.pallas import tpu_sc as plsc

N = 10000
E = 160000
D = 128
NB = 32

NCORE = 2
NSUB = 16
EPT = E // NSUB
CH = 80
NCH = EPT // CH
RPT = 624
LAST = N - (NSUB - 1) * RPT


def _mlp_body(x_ref, ws1, bs1, ws2, bs2, wv1, bv1, wv2, bv2, out_ref):
    x = x_ref[...]
    hs = jax.nn.silu(jnp.dot(x, ws1[...], preferred_element_type=jnp.float32) + bs1[...])
    out_ref[0] = jnp.dot(hs, ws2[...], preferred_element_type=jnp.float32) + bs2[...]
    hv = jax.nn.silu(jnp.dot(x, wv1[...], preferred_element_type=jnp.float32) + bv1[...])
    out_ref[1] = jnp.dot(hv, wv2[...], preferred_element_type=jnp.float32) + bv2[...]


def _edge_mlps(el, Ws1, bs1, Ws2, bs2, Wv1, bv1, Wv2, bv2):
    BE = 16000
    full = lambda shp: pl.BlockSpec(shp, lambda i: (0,) * len(shp))
    return pl.pallas_call(
        _mlp_body,
        grid=(E // BE,),
        in_specs=[
            pl.BlockSpec((BE, NB), lambda i: (i, 0)),
            full((NB, D)), full((1, D)), full((D, D)), full((1, D)),
            full((NB, D)), full((1, D)), full((D, D)), full((1, D)),
        ],
        out_specs=pl.BlockSpec((2, BE, D), lambda i: (0, i, 0)),
        out_shape=jax.ShapeDtypeStruct((2, E, D), jnp.float32),
    )(el, Ws1, bs1.reshape(1, D), Ws2, bs2.reshape(1, D),
      Wv1, bv1.reshape(1, D), Wv2, bv2.reshape(1, D))


def _sc_body(tabs, tabv, wst, colf, rowf, zrows, outs, outv,
             accum, icA, icB, irA, irB, gbA, gbB, rA, wA, rB, wB,
             sIA, sIB, sAg, sAw, sBg, sBw):
    c = lax.axis_index("c")
    s = lax.axis_index("s")

    def idx_issue(g, icbuf, irbuf, si):
        base = s * EPT + g * CH
        pltpu.async_copy(colf.at[pl.ds(base, CH)], icbuf, si)
        pltpu.async_copy(rowf.at[pl.ds(base, CH)], irbuf, si)

    def idx_wait(g, icbuf, irbuf, si):
        base = s * EPT + g * CH
        pltpu.make_async_copy(colf.at[pl.ds(base, CH)], icbuf, si).wait()
        pltpu.make_async_copy(rowf.at[pl.ds(base, CH)], irbuf, si).wait()

    def run_phase(tab, mult, chb, wbase):

        def bias(icbuf, gbuf):
            for k in range(CH // 16):
                gbuf[pl.ds(k * 16, 16)] = icbuf[pl.ds(k * 16, 16)] * mult + chb

        def data_issue(g, gbuf, rbuf, wbuf, sg, sw):
            pltpu.async_copy(tab.at[gbuf], rbuf, sg)
            pltpu.async_copy(wst.at[pl.ds(wbase + g * CH, CH)], wbuf, sw)

        def data_wait(g, gbuf, rbuf, wbuf, sg, sw):
            pltpu.make_async_copy(tab.at[gbuf], rbuf, sg).wait()
            pltpu.make_async_copy(wst.at[pl.ds(wbase + g * CH, CH)], wbuf, sw).wait()

        def mul_scatter(irbuf, rbuf, wbuf, pre_issue=None):
            def mrow(r, _):
                for j in range(D // 16):
                    sl = pl.ds(j * 16, 16)
                    rbuf[r, sl] = rbuf[r, sl] * wbuf[r, sl]
                return 0
            lax.fori_loop(0, CH, mrow, 0)
            rows16 = [irbuf[pl.ds(k * 16, 16)] for k in range(CH // 16)]
            if pre_issue is not None:
                pre_issue()
            for k in range(CH // 16):
                pltpu.sync_copy(rbuf.at[pl.ds(k * 16, 16)],
                                accum.at[rows16[k]], add=True)

        idx_issue(0, icA, irA, sIA)
        idx_wait(0, icA, irA, sIA)
        bias(icA, gbA)
        data_issue(0, gbA, rA, wA, sAg, sAw)
        idx_issue(1, icB, irB, sIB)

        def chunk_pair(i, _):
            g0 = 2 * i
            idx_wait(g0 + 1, icB, irB, sIB)
            bias(icB, gbB)
            data_issue(g0 + 1, gbB, rB, wB, sBg, sBw)

            data_wait(g0, gbA, rA, wA, sAg, sAw)
            mul_scatter(irA, rA, wA,
                        pre_issue=lambda: idx_issue(g0 + 2, icA, irA, sIA))

            idx_wait(g0 + 2, icA, irA, sIA)
            bias(icA, gbA)
            data_issue(g0 + 2, gbA, rA, wA, sAg, sAw)

            data_wait(g0 + 1, gbB, rB, wB, sBg, sBw)

            def refill_b():
                @pl.when(i < NCH // 2 - 1)
                def _():
                    idx_issue(g0 + 3, icB, irB, sIB)

            mul_scatter(irB, rB, wB, pre_issue=refill_b)
            return 0
        lax.fori_loop(0, NCH // 2, chunk_pair, 0)

        data_wait(NCH - 1, gbA, rA, wA, sAg, sAw)
        mul_scatter(irA, rA, wA)

    def writeout_s():
        @pl.when(s < NSUB - 1)
        def _():
            pltpu.sync_copy(accum.at[pl.ds(s * RPT, RPT)],
                            outs.at[pl.ds(s * RPT, RPT)])

        @pl.when(s == NSUB - 1)
        def _():
            pltpu.sync_copy(accum.at[pl.ds((NSUB - 1) * RPT, LAST)],
                            outs.at[pl.ds((NSUB - 1) * RPT, LAST)])

    def writeout_v(ch):
        dbase = ch * N

        @pl.when(s < NSUB - 1)
        def _():
            pltpu.sync_copy(accum.at[pl.ds(s * RPT, RPT)],
                            outv.at[pl.ds(dbase + s * RPT, RPT)])

        @pl.when(s == NSUB - 1)
        def _():
            pltpu.sync_copy(accum.at[pl.ds((NSUB - 1) * RPT, LAST)],
                            outv.at[pl.ds(dbase + (NSUB - 1) * RPT, LAST)])

    def zero_accum():
        @pl.when(s < NSUB - 1)
        def _():
            pltpu.sync_copy(zrows.at[pl.ds(0, RPT)], accum.at[pl.ds(s * RPT, RPT)])

        @pl.when(s == NSUB - 1)
        def _():
            pltpu.sync_copy(zrows, accum.at[pl.ds((NSUB - 1) * RPT, LAST)])

    zero_accum()
    plsc.subcore_barrier()

    @pl.when(c == 0)
    def _():
        run_phase(tabs, 1, 0 * c, s * EPT)

    @pl.when(c == 1)
    def _():
        run_phase(tabv, 3, 0 * c + 1, E + s * EPT)

    plsc.subcore_barrier()

    @pl.when(c == 0)
    def _():
        writeout_s()

    @pl.when(c == 1)
    def _():
        writeout_v(0 * c + 1)

    plsc.subcore_barrier()

    zero_accum()
    plsc.subcore_barrier()

    run_phase(tabv, 3, 2 * c, E + s * EPT)

    plsc.subcore_barrier()
    writeout_v(2 * c)
    plsc.subcore_barrier()


_sc_gather_scatter = functools.partial(
    pl.kernel,
    out_type=(jax.ShapeDtypeStruct((N, D), jnp.float32),
              jax.ShapeDtypeStruct((3 * N, D), jnp.float32)),
    mesh=plsc.VectorSubcoreMesh(core_axis_name="c", subcore_axis_name="s"),
    scratch_types=[
        pltpu.VMEM_SHARED((N, D), jnp.float32),
        pltpu.VMEM((CH,), jnp.int32),
        pltpu.VMEM((CH,), jnp.int32),
        pltpu.VMEM((CH,), jnp.int32),
        pltpu.VMEM((CH,), jnp.int32),
        pltpu.VMEM((CH,), jnp.int32),
        pltpu.VMEM((CH,), jnp.int32),
        pltpu.VMEM((CH, D), jnp.float32),
        pltpu.VMEM((CH, D), jnp.float32),
        pltpu.VMEM((CH, D), jnp.float32),
        pltpu.VMEM((CH, D), jnp.float32),
        pltpu.SemaphoreType.DMA,
        pltpu.SemaphoreType.DMA,
        pltpu.SemaphoreType.DMA,
        pltpu.SemaphoreType.DMA,
        pltpu.SemaphoreType.DMA,
        pltpu.SemaphoreType.DMA,
    ],
)(_sc_body)


def kernel(scalar, vector, edge_sh, edge_length, edge_index, Ws1, bs1, Ws2, bs2, Wv1, bv1, Wv2, bv2):
    del edge_sh
    wst = _edge_mlps(edge_length, Ws1, bs1, Ws2, bs2, Wv1, bv1, Wv2, bv2)
    rowf = edge_index[0].astype(jnp.int32)
    colf = edge_index[1].astype(jnp.int32)
    tabv = vector.reshape(N * 3, D)
    zrows = jnp.zeros((LAST, D), jnp.float32)
    out_scalar, outv = _sc_gather_scatter(scalar, tabv, wst.reshape(2 * E, D),
                                          colf, rowf, zrows)
    return out_scalar, jnp.moveaxis(outv.reshape(3, N, D), 0, 1)


# speedup vs baseline: 1.0735x; 1.0090x over previous
import functools

import jax
import jax.numpy as jnp
from jax import lax
from jax.experimental import pallas as pl
from jax.experimental.pallas import tpu as pltpu
from jax.experimental.pallas import tpu_sc as plsc

N = 10000
E = 160000
D = 128
NB = 32

NCORE = 2
NSUB = 16
EPT = E // NSUB
CH = 80
NCH = EPT // CH
RPT = 624
LAST = N - (NSUB - 1) * RPT


def _mlp_body(x_ref, ws1, bs1, ws2, bs2, wv1, bv1, wv2, bv2, out_ref):
    x = x_ref[...]
    hs = jax.nn.silu(jnp.dot(x, ws1[...], preferred_element_type=jnp.float32) + bs1[...])
    out_ref[0] = jnp.dot(hs, ws2[...], preferred_element_type=jnp.float32) + bs2[...]
    hv = jax.nn.silu(jnp.dot(x, wv1[...], preferred_element_type=jnp.float32) + bv1[...])
    out_ref[1] = jnp.dot(hv, wv2[...], preferred_element_type=jnp.float32) + bv2[...]


def _edge_mlps(el, Ws1, bs1, Ws2, bs2, Wv1, bv1, Wv2, bv2):
    BE = 16000
    full = lambda shp: pl.BlockSpec(shp, lambda i: (0,) * len(shp))
    return pl.pallas_call(
        _mlp_body,
        grid=(E // BE,),
        in_specs=[
            pl.BlockSpec((BE, NB), lambda i: (i, 0)),
            full((NB, D)), full((1, D)), full((D, D)), full((1, D)),
            full((NB, D)), full((1, D)), full((D, D)), full((1, D)),
        ],
        out_specs=pl.BlockSpec((2, BE, D), lambda i: (0, i, 0)),
        out_shape=jax.ShapeDtypeStruct((2, E, D), jnp.float32),
    )(el, Ws1, bs1.reshape(1, D), Ws2, bs2.reshape(1, D),
      Wv1, bv1.reshape(1, D), Wv2, bv2.reshape(1, D))


def _sc_body(tabs, tabv, wst, eix, zrows, outs, outv,
             accum, icA, icB, irA, irB, gbA, gbB, rA, wA, rB, wB,
             sIA, sIB, sAg, sAw, sBg, sBw):
    c = lax.axis_index("c")
    s = lax.axis_index("s")

    def idx_issue(g, icbuf, irbuf, si):
        base = s * EPT + g * CH
        pltpu.async_copy(eix.at[pl.ds(E + base, CH)], icbuf, si)
        pltpu.async_copy(eix.at[pl.ds(base, CH)], irbuf, si)

    def idx_wait(g, icbuf, irbuf, si):
        base = s * EPT + g * CH
        pltpu.make_async_copy(eix.at[pl.ds(E + base, CH)], icbuf, si).wait()
        pltpu.make_async_copy(eix.at[pl.ds(base, CH)], irbuf, si).wait()

    def run_phase(tab, mult, chb, wbase):

        def bias(icbuf, gbuf):
            for k in range(CH // 16):
                gbuf[pl.ds(k * 16, 16)] = icbuf[pl.ds(k * 16, 16)] * mult + chb

        def data_issue(g, gbuf, rbuf, wbuf, sg, sw):
            pltpu.async_copy(tab.at[gbuf], rbuf, sg)
            pltpu.async_copy(wst.at[pl.ds(wbase + g * CH, CH)], wbuf, sw)

        def data_wait(g, gbuf, rbuf, wbuf, sg, sw):
            pltpu.make_async_copy(tab.at[gbuf], rbuf, sg).wait()
            pltpu.make_async_copy(wst.at[pl.ds(wbase + g * CH, CH)], wbuf, sw).wait()

        def mul_scatter(irbuf, rbuf, wbuf, pre_issue=None):
            def mrow(r, _):
                for j in range(D // 16):
                    sl = pl.ds(j * 16, 16)
                    rbuf[r, sl] = rbuf[r, sl] * wbuf[r, sl]
                return 0
            lax.fori_loop(0, CH, mrow, 0)
            rows16 = [irbuf[pl.ds(k * 16, 16)] for k in range(CH // 16)]
            if pre_issue is not None:
                pre_issue()
            for k in range(CH // 16):
                pltpu.sync_copy(rbuf.at[pl.ds(k * 16, 16)],
                                accum.at[rows16[k]], add=True)

        idx_issue(0, icA, irA, sIA)
        idx_wait(0, icA, irA, sIA)
        bias(icA, gbA)
        data_issue(0, gbA, rA, wA, sAg, sAw)
        idx_issue(1, icB, irB, sIB)

        def chunk_pair(i, _):
            g0 = 2 * i
            idx_wait(g0 + 1, icB, irB, sIB)
            bias(icB, gbB)
            data_issue(g0 + 1, gbB, rB, wB, sBg, sBw)

            data_wait(g0, gbA, rA, wA, sAg, sAw)
            mul_scatter(irA, rA, wA,
                        pre_issue=lambda: idx_issue(g0 + 2, icA, irA, sIA))

            idx_wait(g0 + 2, icA, irA, sIA)
            bias(icA, gbA)
            data_issue(g0 + 2, gbA, rA, wA, sAg, sAw)

            data_wait(g0 + 1, gbB, rB, wB, sBg, sBw)

            def refill_b():
                @pl.when(i < NCH // 2 - 1)
                def _():
                    idx_issue(g0 + 3, icB, irB, sIB)

            mul_scatter(irB, rB, wB, pre_issue=refill_b)
            return 0
        lax.fori_loop(0, NCH // 2, chunk_pair, 0)

        data_wait(NCH - 1, gbA, rA, wA, sAg, sAw)
        mul_scatter(irA, rA, wA)

    def writeout_s():
        @pl.when(s < NSUB - 1)
        def _():
            pltpu.sync_copy(accum.at[pl.ds(s * RPT, RPT)],
                            outs.at[pl.ds(s * RPT, RPT)])

        @pl.when(s == NSUB - 1)
        def _():
            pltpu.sync_copy(accum.at[pl.ds((NSUB - 1) * RPT, LAST)],
                            outs.at[pl.ds((NSUB - 1) * RPT, LAST)])

    def writeout_v(ch):
        dbase = ch * N

        @pl.when(s < NSUB - 1)
        def _():
            pltpu.sync_copy(accum.at[pl.ds(s * RPT, RPT)],
                            outv.at[pl.ds(dbase + s * RPT, RPT)])

        @pl.when(s == NSUB - 1)
        def _():
            pltpu.sync_copy(accum.at[pl.ds((NSUB - 1) * RPT, LAST)],
                            outv.at[pl.ds(dbase + (NSUB - 1) * RPT, LAST)])

    def zero_accum():
        @pl.when(s < NSUB - 1)
        def _():
            pltpu.sync_copy(zrows.at[pl.ds(0, RPT)], accum.at[pl.ds(s * RPT, RPT)])

        @pl.when(s == NSUB - 1)
        def _():
            pltpu.sync_copy(zrows, accum.at[pl.ds((NSUB - 1) * RPT, LAST)])

    zero_accum()
    plsc.subcore_barrier()

    @pl.when(c == 0)
    def _():
        run_phase(tabs, 1, 0 * c, s * EPT)

    @pl.when(c == 1)
    def _():
        run_phase(tabv, 3, 0 * c + 1, E + s * EPT)

    plsc.subcore_barrier()

    @pl.when(c == 0)
    def _():
        writeout_s()

    @pl.when(c == 1)
    def _():
        writeout_v(0 * c + 1)

    plsc.subcore_barrier()

    zero_accum()
    plsc.subcore_barrier()

    run_phase(tabv, 3, 2 * c, E + s * EPT)

    plsc.subcore_barrier()
    writeout_v(2 * c)
    plsc.subcore_barrier()


_sc_gather_scatter = functools.partial(
    pl.kernel,
    out_type=(jax.ShapeDtypeStruct((N, D), jnp.float32),
              jax.ShapeDtypeStruct((3 * N, D), jnp.float32)),
    mesh=plsc.VectorSubcoreMesh(core_axis_name="c", subcore_axis_name="s"),
    scratch_types=[
        pltpu.VMEM_SHARED((N, D), jnp.float32),
        pltpu.VMEM((CH,), jnp.int32),
        pltpu.VMEM((CH,), jnp.int32),
        pltpu.VMEM((CH,), jnp.int32),
        pltpu.VMEM((CH,), jnp.int32),
        pltpu.VMEM((CH,), jnp.int32),
        pltpu.VMEM((CH,), jnp.int32),
        pltpu.VMEM((CH, D), jnp.float32),
        pltpu.VMEM((CH, D), jnp.float32),
        pltpu.VMEM((CH, D), jnp.float32),
        pltpu.VMEM((CH, D), jnp.float32),
        pltpu.SemaphoreType.DMA,
        pltpu.SemaphoreType.DMA,
        pltpu.SemaphoreType.DMA,
        pltpu.SemaphoreType.DMA,
        pltpu.SemaphoreType.DMA,
        pltpu.SemaphoreType.DMA,
    ],
)(_sc_body)


def kernel(scalar, vector, edge_sh, edge_length, edge_index, Ws1, bs1, Ws2, bs2, Wv1, bv1, Wv2, bv2):
    del edge_sh
    wst = _edge_mlps(edge_length, Ws1, bs1, Ws2, bs2, Wv1, bv1, Wv2, bv2)
    eix = edge_index.astype(jnp.int32).reshape(2 * E)
    tabv = vector.reshape(N * 3, D)
    zrows = jnp.zeros((LAST, D), jnp.float32)
    out_scalar, outv = _sc_gather_scatter(scalar, tabv, wst.reshape(2 * E, D),
                                          eix, zrows)
    return out_scalar, jnp.moveaxis(outv.reshape(3, N, D), 0, 1)


# speedup vs baseline: 1.0799x; 1.0060x over previous
import functools

import jax
import jax.numpy as jnp
from jax import lax
from jax.experimental import pallas as pl
from jax.experimental.pallas import tpu as pltpu
from jax.experimental.pallas import tpu_sc as plsc

N = 10000
E = 160000
D = 128
NB = 32

NCORE = 2
NSUB = 16
EPT = E // NSUB
CH = 80
NCH = EPT // CH
RPT = 624
LAST = N - (NSUB - 1) * RPT


def _mlp_body(x_ref, ws1, bs1, ws2, bs2, wv1, bv1, wv2, bv2, os_ref, ov_ref):
    x = x_ref[...]
    hs = jax.nn.silu(jnp.dot(x, ws1[...], preferred_element_type=jnp.float32) + bs1[...])
    os_ref[...] = jnp.dot(hs, ws2[...], preferred_element_type=jnp.float32) + bs2[...]
    hv = jax.nn.silu(jnp.dot(x, wv1[...], preferred_element_type=jnp.float32) + bv1[...])
    ov_ref[...] = jnp.dot(hv, wv2[...], preferred_element_type=jnp.float32) + bv2[...]


def _edge_mlps(el, Ws1, bs1, Ws2, bs2, Wv1, bv1, Wv2, bv2):
    BE = 16000
    full = lambda shp: pl.BlockSpec(shp, lambda i: (0,) * len(shp))
    return pl.pallas_call(
        _mlp_body,
        grid=(E // BE,),
        in_specs=[
            pl.BlockSpec((BE, NB), lambda i: (i, 0)),
            full((NB, D)), full((1, D)), full((D, D)), full((1, D)),
            full((NB, D)), full((1, D)), full((D, D)), full((1, D)),
        ],
        out_specs=[pl.BlockSpec((BE, D), lambda i: (i, 0)),
                   pl.BlockSpec((BE, D), lambda i: (i, 0))],
        out_shape=(jax.ShapeDtypeStruct((E, D), jnp.float32),
                   jax.ShapeDtypeStruct((E, D), jnp.float32)),
    )(el, Ws1, bs1.reshape(1, D), Ws2, bs2.reshape(1, D),
      Wv1, bv1.reshape(1, D), Wv2, bv2.reshape(1, D))


def _sc_body(tabs, tabv, ws, wv, eix, zrows, outs, outv,
             accum, icA, icB, irA, irB, gbA, gbB, rA, wA, rB, wB,
             sIA, sIB, sAg, sAw, sBg, sBw):
    c = lax.axis_index("c")
    s = lax.axis_index("s")

    def idx_issue(g, icbuf, irbuf, si):
        base = s * EPT + g * CH
        pltpu.async_copy(eix.at[pl.ds(E + base, CH)], icbuf, si)
        pltpu.async_copy(eix.at[pl.ds(base, CH)], irbuf, si)

    def idx_wait(g, icbuf, irbuf, si):
        base = s * EPT + g * CH
        pltpu.make_async_copy(eix.at[pl.ds(E + base, CH)], icbuf, si).wait()
        pltpu.make_async_copy(eix.at[pl.ds(base, CH)], irbuf, si).wait()

    def run_phase(tab, mult, chb, wref):
        wbase = s * EPT

        def bias(icbuf, gbuf):
            for k in range(CH // 16):
                gbuf[pl.ds(k * 16, 16)] = icbuf[pl.ds(k * 16, 16)] * mult + chb

        def data_issue(g, gbuf, rbuf, wbuf, sg, sw):
            pltpu.async_copy(tab.at[gbuf], rbuf, sg)
            pltpu.async_copy(wref.at[pl.ds(wbase + g * CH, CH)], wbuf, sw)

        def data_wait(g, gbuf, rbuf, wbuf, sg, sw):
            pltpu.make_async_copy(tab.at[gbuf], rbuf, sg).wait()
            pltpu.make_async_copy(wref.at[pl.ds(wbase + g * CH, CH)], wbuf, sw).wait()

        def mul_scatter(irbuf, rbuf, wbuf, pre_issue=None):
            def mrow(r, _):
                for j in range(D // 16):
                    sl = pl.ds(j * 16, 16)
                    rbuf[r, sl] = rbuf[r, sl] * wbuf[r, sl]
                return 0
            lax.fori_loop(0, CH, mrow, 0)
            rows16 = [irbuf[pl.ds(k * 16, 16)] for k in range(CH // 16)]
            if pre_issue is not None:
                pre_issue()
            for k in range(CH // 16):
                pltpu.sync_copy(rbuf.at[pl.ds(k * 16, 16)],
                                accum.at[rows16[k]], add=True)

        idx_issue(0, icA, irA, sIA)
        idx_wait(0, icA, irA, sIA)
        bias(icA, gbA)
        data_issue(0, gbA, rA, wA, sAg, sAw)
        idx_issue(1, icB, irB, sIB)

        def chunk_pair(i, _):
            g0 = 2 * i
            idx_wait(g0 + 1, icB, irB, sIB)
            bias(icB, gbB)
            data_issue(g0 + 1, gbB, rB, wB, sBg, sBw)

            data_wait(g0, gbA, rA, wA, sAg, sAw)
            mul_scatter(irA, rA, wA,
                        pre_issue=lambda: idx_issue(g0 + 2, icA, irA, sIA))

            idx_wait(g0 + 2, icA, irA, sIA)
            bias(icA, gbA)
            data_issue(g0 + 2, gbA, rA, wA, sAg, sAw)

            data_wait(g0 + 1, gbB, rB, wB, sBg, sBw)

            def refill_b():
                @pl.when(i < NCH // 2 - 1)
                def _():
                    idx_issue(g0 + 3, icB, irB, sIB)

            mul_scatter(irB, rB, wB, pre_issue=refill_b)
            return 0
        lax.fori_loop(0, NCH // 2, chunk_pair, 0)

        data_wait(NCH - 1, gbA, rA, wA, sAg, sAw)
        mul_scatter(irA, rA, wA)

    def writeout_s():
        @pl.when(s < NSUB - 1)
        def _():
            pltpu.sync_copy(accum.at[pl.ds(s * RPT, RPT)],
                            outs.at[pl.ds(s * RPT, RPT)])

        @pl.when(s == NSUB - 1)
        def _():
            pltpu.sync_copy(accum.at[pl.ds((NSUB - 1) * RPT, LAST)],
                            outs.at[pl.ds((NSUB - 1) * RPT, LAST)])

    def writeout_v(ch):
        dbase = ch * N

        @pl.when(s < NSUB - 1)
        def _():
            pltpu.sync_copy(accum.at[pl.ds(s * RPT, RPT)],
                            outv.at[pl.ds(dbase + s * RPT, RPT)])

        @pl.when(s == NSUB - 1)
        def _():
            pltpu.sync_copy(accum.at[pl.ds((NSUB - 1) * RPT, LAST)],
                            outv.at[pl.ds(dbase + (NSUB - 1) * RPT, LAST)])

    def zero_accum():
        @pl.when(s < NSUB - 1)
        def _():
            pltpu.sync_copy(zrows.at[pl.ds(0, RPT)], accum.at[pl.ds(s * RPT, RPT)])

        @pl.when(s == NSUB - 1)
        def _():
            pltpu.sync_copy(zrows, accum.at[pl.ds((NSUB - 1) * RPT, LAST)])

    zero_accum()
    plsc.subcore_barrier()

    @pl.when(c == 0)
    def _():
        run_phase(tabs, 1, 0 * c, ws)

    @pl.when(c == 1)
    def _():
        run_phase(tabv, 3, 0 * c + 1, wv)

    plsc.subcore_barrier()

    @pl.when(c == 0)
    def _():
        writeout_s()

    @pl.when(c == 1)
    def _():
        writeout_v(0 * c + 1)

    plsc.subcore_barrier()

    zero_accum()
    plsc.subcore_barrier()

    run_phase(tabv, 3, 2 * c, wv)

    plsc.subcore_barrier()
    writeout_v(2 * c)
    plsc.subcore_barrier()


_sc_gather_scatter = functools.partial(
    pl.kernel,
    out_type=(jax.ShapeDtypeStruct((N, D), jnp.float32),
              jax.ShapeDtypeStruct((3 * N, D), jnp.float32)),
    mesh=plsc.VectorSubcoreMesh(core_axis_name="c", subcore_axis_name="s"),
    scratch_types=[
        pltpu.VMEM_SHARED((N, D), jnp.float32),
        pltpu.VMEM((CH,), jnp.int32),
        pltpu.VMEM((CH,), jnp.int32),
        pltpu.VMEM((CH,), jnp.int32),
        pltpu.VMEM((CH,), jnp.int32),
        pltpu.VMEM((CH,), jnp.int32),
        pltpu.VMEM((CH,), jnp.int32),
        pltpu.VMEM((CH, D), jnp.float32),
        pltpu.VMEM((CH, D), jnp.float32),
        pltpu.VMEM((CH, D), jnp.float32),
        pltpu.VMEM((CH, D), jnp.float32),
        pltpu.SemaphoreType.DMA,
        pltpu.SemaphoreType.DMA,
        pltpu.SemaphoreType.DMA,
        pltpu.SemaphoreType.DMA,
        pltpu.SemaphoreType.DMA,
        pltpu.SemaphoreType.DMA,
    ],
)(_sc_body)


def kernel(scalar, vector, edge_sh, edge_length, edge_index, Ws1, bs1, Ws2, bs2, Wv1, bv1, Wv2, bv2):
    del edge_sh
    ws, wv = _edge_mlps(edge_length, Ws1, bs1, Ws2, bs2, Wv1, bv1, Wv2, bv2)
    eix = edge_index.astype(jnp.int32).reshape(2 * E)
    tabv = vector.reshape(N * 3, D)
    zrows = jnp.zeros((LAST, D), jnp.float32)
    out_scalar, outv = _sc_gather_scatter(scalar, tabv, ws, wv, eix, zrows)
    return out_scalar, jnp.moveaxis(outv.reshape(3, N, D), 0, 1)
